# baseline (speedup 1.0000x reference)
import jax
import jax.numpy as jnp
from jax import lax
from jax.experimental import pallas as pl
from jax.experimental.pallas import tpu as pltpu

NEG = -3e38
BF = jnp.bfloat16
F32 = jnp.float32
I32 = jnp.int32


def _ka(hs_ref, w_ref, es_ref, ms_ref, s_ref, abf_ref, gmax_ref, sums_ref):
    i = pl.program_id(0)
    a32 = jnp.dot(hs_ref[...].astype(BF), w_ref[...].astype(BF),
                  preferred_element_type=F32)
    a_bf = a32.astype(BF)

    @pl.when(i == 0)
    def _init():
        gmax_ref[...] = jnp.full(gmax_ref.shape, NEG, F32)
        sums_ref[...] = jnp.zeros(sums_ref.shape, F32)
        abf_ref[...] = a_bf

    sb = lax.dot_general(a_bf, es_ref[...].astype(BF), (((1,), (1,)), ((), ())),
                         preferred_element_type=F32)
    s_ref[...] = sb
    gmax_ref[:, 0:1] = jnp.maximum(gmax_ref[:, 0:1],
                                   jnp.max(sb, axis=1, keepdims=True))
    mb = jnp.where(ms_ref[...] != 0, sb, NEG)
    gmax_ref[:, 1:2] = jnp.maximum(gmax_ref[:, 1:2],
                                   jnp.max(mb, axis=1, keepdims=True))
    lane = lax.broadcasted_iota(I32, (1, 128), 1)
    sums_ref[0:1, :] = jnp.where(lane == i, jnp.sum(sb), sums_ref[0:1, :])
    sums_ref[1:2, :] = jnp.where(lane == i, jnp.sum(sb * sb), sums_ref[1:2, :])


def _mustd(sums, count):
    tot = jnp.sum(sums[0:1, 0:8])
    tsq = jnp.sum(sums[1:2, 0:8])
    mu = tot / count
    var = tsq / count - mu * mu
    return mu, jnp.sqrt(var)


def _kb(s_ref, ms_ref, es_ref, abf_ref, gmax_ref, sums_ref,
        g_ref, pmf_ref, pmi_ref, rstat_ref):
    i = pl.program_id(0)
    n, bm = s_ref.shape

    @pl.when(i == 0)
    def _init():
        rstat_ref[...] = jnp.zeros(rstat_ref.shape, F32)

    mu, std = _mustd(sums_ref[...], jnp.float32(n * bm * pl.num_programs(0)))
    sb = s_ref[...]
    ys = (sb - mu) / std
    maskb = ms_ref[...] != 0
    gmax_y = (gmax_ref[:, 0:1] - mu) / std
    mmax_y = (gmax_ref[:, 1:2] - mu) / std
    e1 = jnp.exp(ys - gmax_y)
    rstat_ref[:, 0:1] += jnp.sum(e1, axis=1, keepdims=True)
    ym = jnp.where(maskb, ys, NEG)
    e2 = jnp.exp(ym - mmax_y)
    rstat_ref[:, 1:2] += jnp.sum(e2, axis=1, keepdims=True)
    cmax = jnp.max(ym, axis=0, keepdims=True)
    sub = lax.broadcasted_iota(I32, (n, bm), 0)
    idxm = jnp.min(jnp.where(ym == cmax, sub, n), axis=0, keepdims=True)
    selm = jnp.sum(jnp.where(sub == idxm, mmax_y, 0.0), axis=0, keepdims=True)
    pmf_ref[0:1, :] = cmax - selm
    pmf_ref[1:2, :] = jnp.where(cmax > -1e30, 1.0, 0.0)
    pmi_ref[0:1, :] = idxm
    tb = lax.dot_general(es_ref[...].astype(BF), abf_ref[...],
                         (((1,), (1,)), ((), ())), preferred_element_type=F32)
    yt = (tb - mu) / std
    rmax = jnp.max(yt, axis=1, keepdims=True)
    lsec = jnp.log(jnp.sum(jnp.exp(yt - rmax), axis=1, keepdims=True)) + rmax
    g_ref[...] = 2.0 * yt - lsec


def _kc(pmi_ref, pmf_ref, rstat_ref, gmax_ref, sums_ref, nout_ref,
        jsafe_ref, expt_ref, lser_ref, e_scr):
    n, m = e_scr.shape
    mu, std = _mustd(sums_ref[...], jnp.float32(n * m))
    gmax_y = (gmax_ref[:, 0:1] - mu) / std
    lse_row = jnp.log(rstat_ref[:, 0:1]) + gmax_y
    subn = lax.broadcasted_iota(I32, (n, n), 0)
    lanen = lax.broadcasted_iota(I32, (n, n), 1)
    lser_lane = jnp.sum(jnp.where(subn == lanen, lse_row, 0.0), axis=0,
                        keepdims=True)
    sub8 = lax.broadcasted_iota(I32, (8, n), 0)
    lser_ref[...] = jnp.where(sub8 == 0, lser_lane, 0.0)
    idx = pmi_ref[0:1, :]
    svs = pmf_ref[0:1, :]
    vcol = pmf_ref[1:2, :]
    den = rstat_ref[:, 1:2]
    subm = lax.broadcasted_iota(I32, (n, m), 0)
    oh = idx == subm
    densel = jnp.sum(jnp.where(oh, den, 0.0), axis=0, keepdims=True)
    prob = jnp.maximum(jnp.exp(svs) / densel, 1e-6)
    val = jnp.where(vcol > 0.5, prob, -jnp.inf)
    e_scr[...] = jnp.where(oh, val, -jnp.inf)
    counts = jnp.sum((e_scr[...] > NEG).astype(I32), axis=1, keepdims=True)
    vcount = jnp.minimum(counts, nout_ref[0, 0])
    lanem = lax.broadcasted_iota(I32, (n, m), 1)
    jcols, ecols = [], []
    for k in range(5):
        e = e_scr[...]
        bestv = jnp.max(e, axis=1, keepdims=True)
        bestm = jnp.min(jnp.where(e == bestv, lanem, m), axis=1, keepdims=True)
        validk = vcount > k
        ecols.append(jnp.where(validk, bestm, -1))
        jcols.append(jnp.where(validk, bestm, 0))
        e_scr[...] = jnp.where(lanem == bestm, -jnp.inf, e)
    jsafe_ref[...] = jnp.concatenate(
        jcols + [vcount, jnp.zeros((n, 2), I32)], axis=1)
    expt_ref[...] = jnp.concatenate(
        ecols + [jnp.full((n, 3), -1, I32)], axis=1)


def _kd(jref, g0, g1, g2, g3, g4, lser_ref, out_ref):
    i = pl.program_id(0)
    lser = lser_ref[0:1, :]
    vc = jref[i, 5]
    rows = []
    for k, gk in enumerate((g0, g1, g2, g3, g4)):
        fac = jnp.where(vc > k, 1.0, 0.0)
        rows.append((gk[0] - lser) * fac)
    out_ref[...] = jnp.concatenate(rows, axis=0)[None]


def kernel(hs, es, mask, n_output, W):
    N, dim = hs.shape
    M = es.shape[0]
    K = 5
    BM = 512
    nblk = M // BM
    mask8 = mask.astype(jnp.int8)

    s, abf, gmax, sums = pl.pallas_call(
        _ka,
        grid=(nblk,),
        in_specs=[
            pl.BlockSpec((N, dim), lambda i: (0, 0)),
            pl.BlockSpec((dim, dim), lambda i: (0, 0)),
            pl.BlockSpec((BM, dim), lambda i: (i, 0)),
            pl.BlockSpec((N, BM), lambda i: (0, i)),
        ],
        out_specs=[
            pl.BlockSpec((N, BM), lambda i: (0, i)),
            pl.BlockSpec((N, dim), lambda i: (0, 0)),
            pl.BlockSpec((N, 8), lambda i: (0, 0)),
            pl.BlockSpec((8, 128), lambda i: (0, 0)),
        ],
        out_shape=[
            jax.ShapeDtypeStruct((N, M), F32),
            jax.ShapeDtypeStruct((N, dim), BF),
            jax.ShapeDtypeStruct((N, 8), F32),
            jax.ShapeDtypeStruct((8, 128), F32),
        ],
    )(hs, W, es, mask8)

    g, pmf, pmi, rstat = pl.pallas_call(
        _kb,
        grid=(nblk,),
        in_specs=[
            pl.BlockSpec((N, BM), lambda i: (0, i)),
            pl.BlockSpec((N, BM), lambda i: (0, i)),
            pl.BlockSpec((BM, dim), lambda i: (i, 0)),
            pl.BlockSpec((N, dim), lambda i: (0, 0)),
            pl.BlockSpec((N, 8), lambda i: (0, 0)),
            pl.BlockSpec((8, 128), lambda i: (0, 0)),
        ],
        out_specs=[
            pl.BlockSpec((BM, N), lambda i: (i, 0)),
            pl.BlockSpec((8, BM), lambda i: (0, i)),
            pl.BlockSpec((8, BM), lambda i: (0, i)),
            pl.BlockSpec((N, 8), lambda i: (0, 0)),
        ],
        out_shape=[
            jax.ShapeDtypeStruct((M, N), F32),
            jax.ShapeDtypeStruct((8, M), F32),
            jax.ShapeDtypeStruct((8, M), I32),
            jax.ShapeDtypeStruct((N, 8), F32),
        ],
    )(s, mask8, es, abf, gmax, sums)

    nout = jnp.asarray(n_output, I32).reshape(1, 1)
    jsafe, expt, lser = pl.pallas_call(
        _kc,
        grid=(1,),
        in_specs=[
            pl.BlockSpec((8, M), lambda i: (0, 0)),
            pl.BlockSpec((8, M), lambda i: (0, 0)),
            pl.BlockSpec((N, 8), lambda i: (0, 0)),
            pl.BlockSpec((N, 8), lambda i: (0, 0)),
            pl.BlockSpec((8, 128), lambda i: (0, 0)),
            pl.BlockSpec(memory_space=pltpu.SMEM),
        ],
        out_specs=[
            pl.BlockSpec((N, 8), lambda i: (0, 0)),
            pl.BlockSpec((N, 8), lambda i: (0, 0)),
            pl.BlockSpec((8, N), lambda i: (0, 0)),
        ],
        out_shape=[
            jax.ShapeDtypeStruct((N, 8), I32),
            jax.ShapeDtypeStruct((N, 8), I32),
            jax.ShapeDtypeStruct((8, N), F32),
        ],
        scratch_shapes=[pltpu.VMEM((N, M), F32)],
    )(pmi, pmf, rstat, gmax, sums, nout)

    g3d = g.reshape(M, 1, N)
    grid_spec = pltpu.PrefetchScalarGridSpec(
        num_scalar_prefetch=1,
        grid=(N,),
        in_specs=[
            pl.BlockSpec((1, 1, N), lambda i, jr: (jr[i, 0], 0, 0)),
            pl.BlockSpec((1, 1, N), lambda i, jr: (jr[i, 1], 0, 0)),
            pl.BlockSpec((1, 1, N), lambda i, jr: (jr[i, 2], 0, 0)),
            pl.BlockSpec((1, 1, N), lambda i, jr: (jr[i, 3], 0, 0)),
            pl.BlockSpec((1, 1, N), lambda i, jr: (jr[i, 4], 0, 0)),
            pl.BlockSpec((8, N), lambda i, jr: (0, 0)),
        ],
        out_specs=pl.BlockSpec((1, K, N), lambda i, jr: (i, 0, 0)),
    )
    es_out = pl.pallas_call(
        _kd,
        grid_spec=grid_spec,
        out_shape=jax.ShapeDtypeStruct((N, K, N), F32),
    )(jsafe, g3d, g3d, g3d, g3d, g3d, lser)

    expansions = expt[:, :K]
    return (expansions, es_out)


# speedup vs baseline: 3.8064x; 3.8064x over previous
import jax
import jax.numpy as jnp
from jax import lax
from jax.experimental import pallas as pl
from jax.experimental.pallas import tpu as pltpu

NEG = -3e38
BF = jnp.bfloat16
F32 = jnp.float32
I32 = jnp.int32


def _ka(hs_ref, w_ref, es_ref, ms_ref, s_ref, abf_ref, gmax_ref, sums_ref):
    i = pl.program_id(0)
    a32 = jnp.dot(hs_ref[...].astype(BF), w_ref[...].astype(BF),
                  preferred_element_type=F32)
    a_bf = a32.astype(BF)

    @pl.when(i == 0)
    def _init():
        gmax_ref[...] = jnp.full(gmax_ref.shape, NEG, F32)
        sums_ref[...] = jnp.zeros(sums_ref.shape, F32)
        abf_ref[...] = a_bf

    sb = lax.dot_general(a_bf, es_ref[...].astype(BF), (((1,), (1,)), ((), ())),
                         preferred_element_type=F32)
    s_ref[...] = sb
    gmax_ref[:, 0:1] = jnp.maximum(gmax_ref[:, 0:1],
                                   jnp.max(sb, axis=1, keepdims=True))
    mb = jnp.where(ms_ref[...] != 0, sb, NEG)
    gmax_ref[:, 1:2] = jnp.maximum(gmax_ref[:, 1:2],
                                   jnp.max(mb, axis=1, keepdims=True))
    lane = lax.broadcasted_iota(I32, (1, 128), 1)
    sums_ref[0:1, :] = jnp.where(lane == i, jnp.sum(sb), sums_ref[0:1, :])
    sums_ref[1:2, :] = jnp.where(lane == i, jnp.sum(sb * sb), sums_ref[1:2, :])


def _mustd(sums, count):
    tot = jnp.sum(sums[0:1, 0:8])
    tsq = jnp.sum(sums[1:2, 0:8])
    mu = tot / count
    var = tsq / count - mu * mu
    return mu, jnp.sqrt(var)


def _kb(s_ref, ms_ref, es_ref, abf_ref, gmax_ref, sums_ref,
        g_ref, pmf_ref, pmi_ref, rstat_ref):
    i = pl.program_id(0)
    n, bm = s_ref.shape

    @pl.when(i == 0)
    def _init():
        rstat_ref[...] = jnp.zeros(rstat_ref.shape, F32)

    mu, std = _mustd(sums_ref[...], jnp.float32(n * bm * pl.num_programs(0)))
    sb = s_ref[...]
    ys = (sb - mu) / std
    maskb = ms_ref[...] != 0
    gmax_y = (gmax_ref[:, 0:1] - mu) / std
    mmax_y = (gmax_ref[:, 1:2] - mu) / std
    e1 = jnp.exp(ys - gmax_y)
    rstat_ref[:, 0:1] += jnp.sum(e1, axis=1, keepdims=True)
    ym = jnp.where(maskb, ys, NEG)
    e2 = jnp.exp(ym - mmax_y)
    rstat_ref[:, 1:2] += jnp.sum(e2, axis=1, keepdims=True)
    cmax = jnp.max(ym, axis=0, keepdims=True)
    sub = lax.broadcasted_iota(I32, (n, bm), 0)
    idxm = jnp.min(jnp.where(ym == cmax, sub, n), axis=0, keepdims=True)
    selm = jnp.sum(jnp.where(sub == idxm, mmax_y, 0.0), axis=0, keepdims=True)
    pmf_ref[0:1, :] = cmax - selm
    pmf_ref[1:2, :] = jnp.where(cmax > -1e30, 1.0, 0.0)
    pmi_ref[0:1, :] = idxm
    tb = lax.dot_general(es_ref[...].astype(BF), abf_ref[...],
                         (((1,), (1,)), ((), ())), preferred_element_type=F32)
    yt = (tb - mu) / std
    rmax = jnp.max(yt, axis=1, keepdims=True)
    lsec = jnp.log(jnp.sum(jnp.exp(yt - rmax), axis=1, keepdims=True)) + rmax
    g_ref[...] = (2.0 * yt - lsec).astype(BF)


def _kc(pmi_ref, pmf_ref, rstat_ref, gmax_ref, sums_ref, nout_ref,
        jsafe_ref, expt_ref, lser_ref, e_scr):
    n, m = e_scr.shape
    mu, std = _mustd(sums_ref[...], jnp.float32(n * m))
    gmax_y = (gmax_ref[:, 0:1] - mu) / std
    lse_row = jnp.log(rstat_ref[:, 0:1]) + gmax_y
    subn = lax.broadcasted_iota(I32, (n, n), 0)
    lanen = lax.broadcasted_iota(I32, (n, n), 1)
    lser_lane = jnp.sum(jnp.where(subn == lanen, lse_row, 0.0), axis=0,
                        keepdims=True)
    sub8 = lax.broadcasted_iota(I32, (8, n), 0)
    lser_ref[...] = jnp.where(sub8 == 0, lser_lane, 0.0)
    idx = pmi_ref[0:1, :]
    svs = pmf_ref[0:1, :]
    vcol = pmf_ref[1:2, :]
    den = rstat_ref[:, 1:2]
    subm = lax.broadcasted_iota(I32, (n, m), 0)
    oh = idx == subm
    densel = jnp.sum(jnp.where(oh, den, 0.0), axis=0, keepdims=True)
    prob = jnp.maximum(jnp.exp(svs) / densel, 1e-6)
    val = jnp.where(vcol > 0.5, prob, -jnp.inf)
    e_scr[...] = jnp.where(oh, val, -jnp.inf)
    counts = jnp.sum((e_scr[...] > NEG).astype(I32), axis=1, keepdims=True)
    vcount = jnp.minimum(counts, nout_ref[0, 0])
    lanem = lax.broadcasted_iota(I32, (n, m), 1)
    jcols, ecols = [], []
    for k in range(5):
        e = e_scr[...]
        bestv = jnp.max(e, axis=1, keepdims=True)
        bestm = jnp.min(jnp.where(e == bestv, lanem, m), axis=1, keepdims=True)
        validk = vcount > k
        ecols.append(jnp.where(validk, bestm, -1))
        jcols.append(jnp.where(validk, bestm, 0))
        e_scr[...] = jnp.where(lanem == bestm, -jnp.inf, e)
    jsafe_ref[...] = jnp.concatenate(
        jcols + [vcount, jnp.zeros((n, 2), I32)], axis=1)
    expt_ref[...] = jnp.concatenate(
        ecols + [jnp.full((n, 3), -1, I32)], axis=1)


def _kd(jsafe_ref, g_ref, lser_ref, out_ref):
    bi = jsafe_ref.shape[0]
    m = g_ref.shape[0]
    lser = lser_ref[0:1, :]
    gbf = g_ref[...]
    lanem = lax.broadcasted_iota(I32, (bi, m), 1)
    for k in range(5):
        jk = jsafe_ref[:, k:k + 1]
        vk = jsafe_ref[:, 5:6] > k
        oh = jnp.where((lanem == jk) & vk, 1.0, 0.0).astype(BF)
        outk = lax.dot_general(oh, gbf, (((1,), (0,)), ((), ())),
                               preferred_element_type=F32)
        outk = outk - jnp.where(vk, 1.0, 0.0) * lser
        out_ref[:, k:k + 1, :] = outk[:, None, :]


def kernel(hs, es, mask, n_output, W):
    N, dim = hs.shape
    M = es.shape[0]
    K = 5
    BM = 512
    nblk = M // BM
    mask8 = mask.astype(jnp.int8)

    s, abf, gmax, sums = pl.pallas_call(
        _ka,
        grid=(nblk,),
        in_specs=[
            pl.BlockSpec((N, dim), lambda i: (0, 0)),
            pl.BlockSpec((dim, dim), lambda i: (0, 0)),
            pl.BlockSpec((BM, dim), lambda i: (i, 0)),
            pl.BlockSpec((N, BM), lambda i: (0, i)),
        ],
        out_specs=[
            pl.BlockSpec((N, BM), lambda i: (0, i)),
            pl.BlockSpec((N, dim), lambda i: (0, 0)),
            pl.BlockSpec((N, 8), lambda i: (0, 0)),
            pl.BlockSpec((8, 128), lambda i: (0, 0)),
        ],
        out_shape=[
            jax.ShapeDtypeStruct((N, M), F32),
            jax.ShapeDtypeStruct((N, dim), BF),
            jax.ShapeDtypeStruct((N, 8), F32),
            jax.ShapeDtypeStruct((8, 128), F32),
        ],
    )(hs, W, es, mask8)

    g, pmf, pmi, rstat = pl.pallas_call(
        _kb,
        grid=(nblk,),
        in_specs=[
            pl.BlockSpec((N, BM), lambda i: (0, i)),
            pl.BlockSpec((N, BM), lambda i: (0, i)),
            pl.BlockSpec((BM, dim), lambda i: (i, 0)),
            pl.BlockSpec((N, dim), lambda i: (0, 0)),
            pl.BlockSpec((N, 8), lambda i: (0, 0)),
            pl.BlockSpec((8, 128), lambda i: (0, 0)),
        ],
        out_specs=[
            pl.BlockSpec((BM, N), lambda i: (i, 0)),
            pl.BlockSpec((8, BM), lambda i: (0, i)),
            pl.BlockSpec((8, BM), lambda i: (0, i)),
            pl.BlockSpec((N, 8), lambda i: (0, 0)),
        ],
        out_shape=[
            jax.ShapeDtypeStruct((M, N), BF),
            jax.ShapeDtypeStruct((8, M), F32),
            jax.ShapeDtypeStruct((8, M), I32),
            jax.ShapeDtypeStruct((N, 8), F32),
        ],
    )(s, mask8, es, abf, gmax, sums)

    nout = jnp.asarray(n_output, I32).reshape(1, 1)
    jsafe, expt, lser = pl.pallas_call(
        _kc,
        grid=(1,),
        in_specs=[
            pl.BlockSpec((8, M), lambda i: (0, 0)),
            pl.BlockSpec((8, M), lambda i: (0, 0)),
            pl.BlockSpec((N, 8), lambda i: (0, 0)),
            pl.BlockSpec((N, 8), lambda i: (0, 0)),
            pl.BlockSpec((8, 128), lambda i: (0, 0)),
            pl.BlockSpec(memory_space=pltpu.SMEM),
        ],
        out_specs=[
            pl.BlockSpec((N, 8), lambda i: (0, 0)),
            pl.BlockSpec((N, 8), lambda i: (0, 0)),
            pl.BlockSpec((8, N), lambda i: (0, 0)),
        ],
        out_shape=[
            jax.ShapeDtypeStruct((N, 8), I32),
            jax.ShapeDtypeStruct((N, 8), I32),
            jax.ShapeDtypeStruct((8, N), F32),
        ],
        scratch_shapes=[pltpu.VMEM((N, M), F32)],
    )(pmi, pmf, rstat, gmax, sums, nout)

    BI = 128
    es_out = pl.pallas_call(
        _kd,
        grid=(N // BI,),
        in_specs=[
            pl.BlockSpec((BI, 8), lambda i: (i, 0)),
            pl.BlockSpec((M, N), lambda i: (0, 0)),
            pl.BlockSpec((8, N), lambda i: (0, 0)),
        ],
        out_specs=pl.BlockSpec((BI, K, N), lambda i: (i, 0, 0)),
        out_shape=jax.ShapeDtypeStruct((N, K, N), F32),
    )(jsafe, g, lser)

    expansions = expt[:, :K]
    return (expansions, es_out)
